# baseline (speedup 1.0000x reference)
import functools

import jax
import jax.numpy as jnp
from jax import lax
from jax.experimental import pallas as pl
from jax.experimental.pallas import tpu as pltpu
from jax.experimental.pallas import tpu_sc as plsc

EMB = 64
SEQ = 200
BATCH = 16384
LANES = 16
FCH = 8
NIDX = FCH * SEQ
G0 = 104
G1 = SEQ - G0
NC = 2
NS = 16
NW = NC * NS
ROWS_PER_W = BATCH // NW
NCHUNKS = ROWS_PER_W // FCH
NFULL = SEQ // LANES
TAIL_OFF = SEQ - LANES
BB = 64


def _row_count(story_v, sr):
    lane = lax.iota(jnp.int32, 16)
    cnt = lane * 0
    for j in range(NFULL):
        m = story_v[sr, pl.ds(j * LANES, LANES)] != 0
        cnt = cnt + plsc.all_reduce_population_count(m)
    m = story_v[sr, pl.ds(TAIL_OFF, LANES)] != 0
    cnt = cnt + plsc.all_reduce_population_count(
        jnp.logical_and(m, lane >= 8))
    return cnt[0]


def _positions_row(story_v, idx_v, sr, r):
    lane = lax.iota(jnp.int32, 16)
    low8 = lane < 8
    carry = lane * 0
    carry183 = lane * 0
    for j in range(NFULL):
        tok = story_v[sr, pl.ds(j * LANES, LANES)]
        m = tok != 0
        ones = jnp.where(m, 1, 0).astype(jnp.int32)
        csum = plsc.cumsum(ones)
        idx_v[pl.ds(r * SEQ + j * LANES, LANES)] = jnp.where(m, csum + carry, 0)
        if j == NFULL - 1:
            carry183 = carry + plsc.all_reduce_population_count(
                jnp.logical_and(m, low8))
        carry = carry + plsc.all_reduce_population_count(m)
    tok = story_v[sr, pl.ds(TAIL_OFF, LANES)]
    m = tok != 0
    ones = jnp.where(m, 1, 0).astype(jnp.int32)
    csum = plsc.cumsum(ones)
    idx_v[pl.ds(r * SEQ + TAIL_OFF, LANES)] = jnp.where(m, csum + carry183, 0)


def _bcast_body(tab_ref, out_ref):
    out_ref[...] = jnp.broadcast_to(tab_ref[...][None], (BB, SEQ, EMB))


def _broadcast_table(tab):
    return pl.pallas_call(
        _bcast_body,
        grid=(BATCH // BB,),
        in_specs=[pl.BlockSpec((SEQ, EMB), lambda i: (0, 0))],
        out_specs=pl.BlockSpec((BB, SEQ, EMB), lambda i: (i, 0, 0)),
        out_shape=jax.ShapeDtypeStruct((BATCH, SEQ, EMB), jnp.float32),
    )(tab)


def _fix_pad_rows(story2, table, out_flat):
    mesh = plsc.VectorSubcoreMesh(core_axis_name="c", subcore_axis_name="s")

    def stateful(refs):
        story_ref, table_ref, out_ref = refs

        @pl.core_map(
            mesh,
            compiler_params=pltpu.CompilerParams(
                needs_layout_passes=False, use_tc_tiling_on_sc=False
            ),
            scratch_shapes=[
                pltpu.VMEM((FCH, SEQ), jnp.int32),
                pltpu.VMEM((NIDX,), jnp.int32),
                pltpu.VMEM((NIDX, EMB), jnp.float32),
                pltpu.SemaphoreType.DMA,
                pltpu.SemaphoreType.DMA,
            ],
        )
        def _sc_kernel(story_v, idx_v, rows_v, sem_g, sem_o):
            wid = lax.axis_index("s") * NC + lax.axis_index("c")
            base_row = wid * ROWS_PER_W

            def chunk_body(c, _):
                row0 = base_row + c * FCH
                pltpu.sync_copy(story_ref.at[pl.ds(row0, FCH)], story_v)

                conds = []
                for r in range(FCH):
                    cond = _row_count(story_v, r) != SEQ
                    conds.append(cond)

                    @pl.when(cond)
                    def _compute():
                        _positions_row(story_v, idx_v, r, r)

                gathers = []
                for r in range(FCH):
                    @pl.when(conds[r])
                    def _fire():
                        gathers.append((conds[r], pltpu.async_copy(
                            table_ref.at[idx_v.at[pl.ds(r * SEQ, G0)]],
                            rows_v.at[pl.ds(r * SEQ, G0)],
                            sem_g,
                        )))
                        gathers.append((conds[r], pltpu.async_copy(
                            table_ref.at[idx_v.at[pl.ds(r * SEQ + G0, G1)]],
                            rows_v.at[pl.ds(r * SEQ + G0, G1)],
                            sem_g,
                        )))
                for cond, cp in gathers:
                    @pl.when(cond)
                    def _wait_g():
                        cp.wait()

                outs = []
                for r in range(FCH):
                    cp = pltpu.make_async_copy(
                        rows_v.at[pl.ds(r * SEQ, SEQ)],
                        out_ref.at[pl.ds((row0 + r) * SEQ, SEQ)],
                        sem_o,
                    )
                    outs.append((conds[r], cp))

                    @pl.when(conds[r])
                    def _out():
                        cp.start()
                for cond, cp in outs:
                    @pl.when(cond)
                    def _wait_o():
                        cp.wait()
                return ()

            lax.fori_loop(0, NCHUNKS, chunk_body, ())

    _, _, fixed = pl.run_state(stateful)((story2, table, out_flat))
    return fixed


@jax.jit
def _encode(story2, table):
    out = _broadcast_table(lax.slice(table, (1, 0), (SEQ + 1, EMB)))
    out_flat = out.reshape(BATCH * SEQ, EMB)
    fixed = _fix_pad_rows(story2, table, out_flat)
    return fixed.reshape(BATCH, SEQ, EMB)


def kernel(story, table):
    return _encode(story[:, :, 0], table)
